# baseline (speedup 1.0000x reference)
import functools

import jax
import jax.numpy as jnp
from jax import lax
from jax.experimental import pallas as pl
from jax.experimental.pallas import tpu as pltpu
from jax.experimental.pallas import tpu_sc as plsc

N_CLS = 101
CTX_LEN = 77
CLS_PAD = 104
H0 = 48
H1 = 56
CTX_DIM = 512
NUM_CORES = 2
NUM_SUBCORES = 16
NW = NUM_CORES * NUM_SUBCORES
N_UNITS = 2 * CTX_LEN


def kernel(tokenized_prompts, token_embedding):
    idx_t = tokenized_prompts.T
    idxp = jnp.concatenate([idx_t, idx_t[:, -3:]], axis=1).reshape(-1)

    mesh = plsc.VectorSubcoreMesh(core_axis_name="c", subcore_axis_name="s")

    @functools.partial(
        pl.kernel,
        mesh=mesh,
        out_type=jax.ShapeDtypeStruct(
            (CTX_LEN, CLS_PAD, CTX_DIM), token_embedding.dtype),
        scratch_types=[
            pltpu.VMEM((H0,), jnp.int32),
            pltpu.VMEM((H0,), jnp.int32),
            pltpu.VMEM((H1,), jnp.int32),
            pltpu.VMEM((H1,), jnp.int32),
            pltpu.VMEM((H0, CTX_DIM), jnp.float32),
            pltpu.VMEM((H0, CTX_DIM), jnp.float32),
            pltpu.VMEM((H1, CTX_DIM), jnp.float32),
            pltpu.VMEM((H1, CTX_DIM), jnp.float32),
            pltpu.SemaphoreType.DMA,
            pltpu.SemaphoreType.DMA,
            pltpu.SemaphoreType.DMA,
            pltpu.SemaphoreType.DMA,
        ],
    )
    def gather_kernel(table_hbm, idx_hbm, out_hbm,
                      ia0, ia1, ib0, ib1, ba0, ba1, bb0, bb1,
                      sa0, sa1, sb0, sb1):
        wid = lax.axis_index("s") * NUM_CORES + lax.axis_index("c")
        i48 = (ia0, ia1)
        i56 = (ib0, ib1)
        b48 = (ba0, ba1)
        b56 = (bb0, bb1)
        s48 = (sa0, sa1)
        s56 = (sb0, sb1)

        def start(half, j, u):
            t = u // 2
            if half == 0:
                pltpu.sync_copy(
                    idx_hbm.at[pl.ds(CLS_PAD * t, H0)], i48[j])
                cp = pltpu.make_async_copy(
                    table_hbm.at[i48[j]], b48[j], s48[j])
            else:
                pltpu.sync_copy(
                    idx_hbm.at[pl.ds(CLS_PAD * t + H0, H1)], i56[j])
                cp = pltpu.make_async_copy(
                    table_hbm.at[i56[j]], b56[j], s56[j])
            cp.start()
            return cp

        def finish(half, j, u, cp):
            t = u // 2
            cp.wait()
            if half == 0:
                pltpu.sync_copy(b48[j], out_hbm.at[t].at[pl.ds(0, H0)])
            else:
                pltpu.sync_copy(b56[j], out_hbm.at[t].at[pl.ds(H0, H1)])

        def branch(plan, units):
            cp = start(*plan[0], units[0])
            for k in range(3):
                nxt = start(*plan[k + 1], units[k + 1])
                finish(*plan[k], units[k], cp)
                cp = nxt
            finish(*plan[3], units[3], cp)

            @pl.when(wid < 26)
            def _():
                cp4 = start(*plan[4], units[4])
                finish(*plan[4], units[4], cp4)

        @pl.when(wid % 2 == 0)
        def _():
            units = [wid + 32 * k + (k & 1) for k in range(5)]
            branch([(0, 0), (1, 0), (0, 1), (1, 1), (0, 0)], units)

        @pl.when(wid % 2 == 1)
        def _():
            units = [wid + 32 * k - (k & 1) for k in range(5)]
            branch([(1, 0), (0, 0), (1, 1), (0, 1), (1, 0)], units)

    out_t = gather_kernel(token_embedding, idxp)
    return jnp.transpose(out_t, (1, 0, 2))[:N_CLS]


# speedup vs baseline: 1.0379x; 1.0379x over previous
import functools

import jax
import jax.numpy as jnp
from jax import lax
from jax.experimental import pallas as pl
from jax.experimental.pallas import tpu as pltpu
from jax.experimental.pallas import tpu_sc as plsc

N_CLS = 101
CTX_LEN = 77
CLS_PAD = 104
CTX_DIM = 512
NUM_CORES = 2
NUM_SUBCORES = 16
NW = NUM_CORES * NUM_SUBCORES


def kernel(tokenized_prompts, token_embedding):
    idx_t = tokenized_prompts.T
    idxp = jnp.concatenate([idx_t, idx_t[:, -3:]], axis=1).reshape(-1)

    mesh = plsc.VectorSubcoreMesh(core_axis_name="c", subcore_axis_name="s")

    @functools.partial(
        pl.kernel,
        mesh=mesh,
        out_type=jax.ShapeDtypeStruct(
            (CTX_LEN, CLS_PAD, CTX_DIM), token_embedding.dtype),
        scratch_types=[
            pltpu.VMEM((3 * CLS_PAD,), jnp.int32),
            pltpu.VMEM((CLS_PAD, CTX_DIM), jnp.float32),
            pltpu.VMEM((CLS_PAD, CTX_DIM), jnp.float32),
            pltpu.SemaphoreType.DMA,
            pltpu.SemaphoreType.DMA,
        ],
    )
    def gather_kernel(table_hbm, idx_hbm, out_hbm,
                      iv, ma, mb, s0, s1):
        wid = lax.axis_index("s") * NUM_CORES + lax.axis_index("c")
        t0 = 2 * wid + jnp.minimum(wid, 13)
        mains = (ma, mb)
        msems = (s0, s1)

        def start(k):
            b = k % 2
            cp = pltpu.make_async_copy(
                table_hbm.at[iv.at[pl.ds(CLS_PAD * k, CLS_PAD)]],
                mains[b], msems[b])
            cp.start()
            return cp

        def finish(k, cp):
            cp.wait()
            pltpu.sync_copy(mains[k % 2], out_hbm.at[t0 + k])

        def run(nt):
            pltpu.sync_copy(
                idx_hbm.at[pl.ds(CLS_PAD * t0, nt * CLS_PAD)],
                iv.at[pl.ds(0, nt * CLS_PAD)])
            cp = start(0)
            for k in range(nt):
                nxt = start(k + 1) if k + 1 < nt else None
                finish(k, cp)
                cp = nxt

        @pl.when(wid < 13)
        def _():
            run(3)

        @pl.when(wid >= 13)
        def _():
            run(2)

    out_t = gather_kernel(token_embedding, idxp)
    return jnp.transpose(out_t, (1, 0, 2))[:N_CLS]


# speedup vs baseline: 1.0510x; 1.0126x over previous
import functools

import jax
import jax.numpy as jnp
from jax import lax
from jax.experimental import pallas as pl
from jax.experimental.pallas import tpu as pltpu
from jax.experimental.pallas import tpu_sc as plsc

N_CLS = 101
CTX_LEN = 77
CLS_PAD = 104
CTX_DIM = 512
NUM_CORES = 2
NUM_SUBCORES = 16
NW = NUM_CORES * NUM_SUBCORES


def kernel(tokenized_prompts, token_embedding):
    idx_t = tokenized_prompts.T
    idxp = jnp.concatenate([idx_t, idx_t[:, -3:]], axis=1).reshape(-1)

    mesh = plsc.VectorSubcoreMesh(core_axis_name="c", subcore_axis_name="s")

    @functools.partial(
        pl.kernel,
        mesh=mesh,
        out_type=jax.ShapeDtypeStruct(
            (CTX_LEN, CLS_PAD, CTX_DIM), token_embedding.dtype),
        scratch_types=[
            pltpu.VMEM((CLS_PAD,), jnp.int32),
            pltpu.VMEM((CLS_PAD,), jnp.int32),
            pltpu.VMEM((CLS_PAD, CTX_DIM), jnp.float32),
            pltpu.VMEM((CLS_PAD, CTX_DIM), jnp.float32),
            pltpu.SemaphoreType.DMA,
            pltpu.SemaphoreType.DMA,
            pltpu.SemaphoreType.DMA,
            pltpu.SemaphoreType.DMA,
        ],
    )
    def gather_kernel(table_hbm, idx_hbm, out_hbm,
                      i0, i1, ma, mb, s0, s1, s2, s3):
        wid = lax.axis_index("s") * NUM_CORES + lax.axis_index("c")
        t0 = 2 * wid + jnp.minimum(wid, 13)
        ibufs = (i0, i1)
        mains = (ma, mb)
        msems = ((s0, s1), (s2, s3))

        def start(k):
            b = k % 2
            pltpu.sync_copy(
                idx_hbm.at[pl.ds(CLS_PAD * (t0 + k), CLS_PAD)], ibufs[b])
            cpa = pltpu.make_async_copy(
                table_hbm.at[ibufs[b].at[pl.ds(0, 48)]],
                mains[b].at[pl.ds(0, 48)], msems[b][0])
            cpb = pltpu.make_async_copy(
                table_hbm.at[ibufs[b].at[pl.ds(48, 56)]],
                mains[b].at[pl.ds(48, 56)], msems[b][1])
            cpa.start()
            cpb.start()
            return cpa, cpb

        def finish(k, cps):
            cps[0].wait()
            cps[1].wait()
            pltpu.sync_copy(mains[k % 2], out_hbm.at[t0 + k])

        def run(nt):
            cp = start(0)
            for k in range(nt):
                nxt = start(k + 1) if k + 1 < nt else None
                finish(k, cp)
                cp = nxt

        @pl.when(wid < 13)
        def _():
            run(3)

        @pl.when(wid >= 13)
        def _():
            run(2)

    out_t = gather_kernel(token_embedding, idxp)
    return jnp.transpose(out_t, (1, 0, 2))[:N_CLS]
